# baseline (speedup 1.0000x reference)
import functools

import jax
import jax.numpy as jnp
import numpy as np
from jax.experimental import pallas as pl
from jax.experimental.pallas import tpu as pltpu

DELTA_VAR = 1.0
DELTA_DIST = 2.0


def _phase1_body(K, NBLK, data_ref, lab_ref, sums_ref, counts_ref):
    i = pl.program_id(0)
    hb, w = lab_ref.shape
    bsums = jnp.zeros(sums_ref.shape, jnp.float32)
    bcounts = jnp.zeros((1, K), jnp.float32)
    for j in range(hb):
        x = data_ref[:, j, :]
        lab2 = lab_ref[pl.ds(j, 1), :]
        onehot = (jax.lax.broadcasted_iota(jnp.int32, (K, w), 0)
                  == lab2).astype(jnp.float32)
        bsums += jax.lax.dot_general(
            x, onehot, (((1,), (1,)), ((), ())),
            preferred_element_type=jnp.float32)
        bcounts += jnp.sum(onehot, axis=1, keepdims=True).T

    @pl.when(i == 0)
    def _():
        sums_ref[...] = jnp.zeros_like(sums_ref)
        counts_ref[...] = jnp.zeros_like(counts_ref)

    sums_ref[...] += bsums
    counts_ref[...] += bcounts


def _phase2_body(K, NBLK, data_ref, lab_ref, sums_ref, counts_ref, out_ref):
    i = pl.program_id(0)
    hb, w = lab_ref.shape
    D = sums_ref.shape[0]
    centers = sums_ref[...] / counts_ref[...]
    nc = jnp.sum(centers * centers, axis=0, keepdims=True)
    wrow = jnp.concatenate(
        [nc, jnp.full((1, K), -2.0, jnp.float32)], axis=1)
    ones_d = jnp.ones((1, D), jnp.float32)
    hacc = jnp.zeros((1, w), jnp.float32)
    for j in range(hb):
        x = data_ref[:, j, :]
        lab2 = lab_ref[pl.ds(j, 1), :]
        onehot = (jax.lax.broadcasted_iota(jnp.int32, (K, w), 0)
                  == lab2).astype(jnp.float32)
        m = jax.lax.dot_general(
            centers, x, (((0,), (0,)), ((), ())),
            preferred_element_type=jnp.float32)
        stacked = jnp.concatenate([onehot, onehot * m], axis=0)
        s = jax.lax.dot_general(
            wrow, stacked, (((1,), (0,)), ((), ())),
            preferred_element_type=jnp.float32)
        q = jax.lax.dot_general(
            ones_d, x * x, (((1,), (0,)), ((), ())),
            preferred_element_type=jnp.float32)
        norm2 = jnp.maximum(q + s, 0.0)
        norm = jnp.sqrt(norm2)
        h = jnp.maximum(norm - DELTA_VAR, 0.0)
        hacc += h * h
    var_b = jnp.sum(hacc)

    @pl.when(i == 0)
    def _():
        out_ref[0, 0] = 0.0

    out_ref[0, 0] += var_b / K

    @pl.when(i == NBLK - 1)
    def _():
        delta_reg = float(np.sqrt(centers.shape[0]))
        n2 = jnp.sum(centers * centers, axis=0)
        gram = jax.lax.dot_general(
            centers, centers, (((0,), (0,)), ((), ())),
            preferred_element_type=jnp.float32)
        sq = jnp.maximum(n2[:, None] + n2[None, :] - 2.0 * gram, 0.0)
        eye = jnp.eye(K, dtype=jnp.float32)
        cnorm = jnp.sqrt(sq + eye)
        hinge = (jnp.maximum(2.0 * DELTA_DIST - cnorm, 0.0) ** 2) * (1.0 - eye)
        dist_term = jnp.sum(hinge) / (K * (K - 1))
        reg_term = jnp.sum(jnp.maximum(jnp.sqrt(n2) - delta_reg, 0.0)) / K
        out_ref[0, 0] += dist_term + reg_term


def kernel(data, labels, cluster_ids):
    D, H, W = data.shape
    K = cluster_ids.shape[0]
    HB = 16
    NBLK = H // HB

    sums, counts = pl.pallas_call(
        functools.partial(_phase1_body, K, NBLK),
        grid=(NBLK,),
        in_specs=[
            pl.BlockSpec((D, HB, W), lambda i: (0, i, 0)),
            pl.BlockSpec((HB, W), lambda i: (i, 0)),
        ],
        out_specs=[
            pl.BlockSpec((D, K), lambda i: (0, 0)),
            pl.BlockSpec((1, K), lambda i: (0, 0)),
        ],
        out_shape=[
            jax.ShapeDtypeStruct((D, K), jnp.float32),
            jax.ShapeDtypeStruct((1, K), jnp.float32),
        ],
    )(data, labels)

    out = pl.pallas_call(
        functools.partial(_phase2_body, K, NBLK),
        grid=(NBLK,),
        in_specs=[
            pl.BlockSpec((D, HB, W), lambda i: (0, i, 0)),
            pl.BlockSpec((HB, W), lambda i: (i, 0)),
            pl.BlockSpec((D, K), lambda i: (0, 0)),
            pl.BlockSpec((1, K), lambda i: (0, 0)),
        ],
        out_specs=pl.BlockSpec(memory_space=pltpu.SMEM),
        out_shape=jax.ShapeDtypeStruct((1, 1), jnp.float32),
    )(data, labels, sums, counts)

    return out[0, 0]


# speedup vs baseline: 1.4726x; 1.4726x over previous
import functools

import jax
import jax.numpy as jnp
import numpy as np
from jax.experimental import pallas as pl
from jax.experimental.pallas import tpu as pltpu

DELTA_VAR = 1.0
DELTA_DIST = 2.0


def _phase1_body(K, NBLK, data_ref, lab_ref, sums_ref, counts_ref):
    i = pl.program_id(0)
    hb, w = lab_ref.shape
    D = sums_ref.shape[0]
    x = data_ref[...].reshape(D, hb * w)
    lab2 = lab_ref[...].reshape(1, hb * w)
    onehot = (jax.lax.broadcasted_iota(jnp.int32, (K, hb * w), 0)
              == lab2).astype(jnp.float32)
    bsums = jax.lax.dot_general(
        x, onehot, (((1,), (1,)), ((), ())),
        preferred_element_type=jnp.float32)
    bcounts = jnp.sum(onehot, axis=1, keepdims=True).T

    @pl.when(i == 0)
    def _():
        sums_ref[...] = jnp.zeros_like(sums_ref)
        counts_ref[...] = jnp.zeros_like(counts_ref)

    sums_ref[...] += bsums
    counts_ref[...] += bcounts


def _phase2_body(K, NBLK, data_ref, lab_ref, sums_ref, counts_ref, out_ref):
    i = pl.program_id(0)
    hb, w = lab_ref.shape
    D = sums_ref.shape[0]
    centers = sums_ref[...] / counts_ref[...]
    x = data_ref[...].reshape(D, hb * w)
    lab2 = lab_ref[...].reshape(1, hb * w)
    onehot = (jax.lax.broadcasted_iota(jnp.int32, (K, hb * w), 0)
              == lab2).astype(jnp.float32)
    c_sel = jax.lax.dot_general(
        centers, onehot, (((1,), (0,)), ((), ())),
        preferred_element_type=jnp.float32)
    diff = x - c_sel
    norm2 = jnp.sum(diff * diff, axis=0, keepdims=True)
    norm = jnp.sqrt(norm2)
    h = jnp.maximum(norm - DELTA_VAR, 0.0)
    var_b = jnp.sum(h * h)

    @pl.when(i == 0)
    def _():
        out_ref[0, 0] = 0.0

    out_ref[0, 0] += var_b / K

    @pl.when(i == NBLK - 1)
    def _():
        delta_reg = float(np.sqrt(centers.shape[0]))
        n2 = jnp.sum(centers * centers, axis=0)
        gram = jax.lax.dot_general(
            centers, centers, (((0,), (0,)), ((), ())),
            preferred_element_type=jnp.float32)
        sq = jnp.maximum(n2[:, None] + n2[None, :] - 2.0 * gram, 0.0)
        eye = jnp.eye(K, dtype=jnp.float32)
        cnorm = jnp.sqrt(sq + eye)
        hinge = (jnp.maximum(2.0 * DELTA_DIST - cnorm, 0.0) ** 2) * (1.0 - eye)
        dist_term = jnp.sum(hinge) / (K * (K - 1))
        reg_term = jnp.sum(jnp.maximum(jnp.sqrt(n2) - delta_reg, 0.0)) / K
        out_ref[0, 0] += dist_term + reg_term


def kernel(data, labels, cluster_ids):
    D, H, W = data.shape
    K = cluster_ids.shape[0]
    HB = 16
    NBLK = H // HB

    sums, counts = pl.pallas_call(
        functools.partial(_phase1_body, K, NBLK),
        grid=(NBLK,),
        in_specs=[
            pl.BlockSpec((D, HB, W), lambda i: (0, i, 0)),
            pl.BlockSpec((HB, W), lambda i: (i, 0)),
        ],
        out_specs=[
            pl.BlockSpec((D, K), lambda i: (0, 0)),
            pl.BlockSpec((1, K), lambda i: (0, 0)),
        ],
        out_shape=[
            jax.ShapeDtypeStruct((D, K), jnp.float32),
            jax.ShapeDtypeStruct((1, K), jnp.float32),
        ],
    )(data, labels)

    out = pl.pallas_call(
        functools.partial(_phase2_body, K, NBLK),
        grid=(NBLK,),
        in_specs=[
            pl.BlockSpec((D, HB, W), lambda i: (0, i, 0)),
            pl.BlockSpec((HB, W), lambda i: (i, 0)),
            pl.BlockSpec((D, K), lambda i: (0, 0)),
            pl.BlockSpec((1, K), lambda i: (0, 0)),
        ],
        out_specs=pl.BlockSpec(memory_space=pltpu.SMEM),
        out_shape=jax.ShapeDtypeStruct((1, 1), jnp.float32),
    )(data, labels, sums, counts)

    return out[0, 0]
